# Initial kernel scaffold; baseline (speedup 1.0000x reference)
import jax
import jax.numpy as jnp
from jax.experimental import pallas as pl


def kernel(x, edge_index, W0, b0, W1, b1, W2, b2, eps0, eps1, eps2, g0, be0, g1, be1):
    raise NotImplementedError("write your pallas kernel here")


# speedup vs baseline: 7.0551x; 7.0551x over previous
import functools

import jax
import jax.numpy as jnp
from jax import lax
from jax.experimental import pallas as pl
from jax.experimental.pallas import tpu as pltpu
from jax.experimental.pallas import tpu_sc as plsc

N = 10000
E = 320000
DIN = 128
DH = 128
DOUT = 40
BN_EPS = 1e-5

NC = 2
NS = 16
NW = NC * NS
EW = E // NW
CHUNK = 80
NCH = EW // CHUNK
NPAD = 10240
RPT = NPAD // NS


def _sc_agg_body(t_hbm, src_hbm, dst_hbm, zero_hbm, out_hbm,
                 srcv, dstv, rows, acc, sem):
    cid = lax.axis_index("c")
    sid = lax.axis_index("s")
    wid = sid * NC + cid

    pltpu.sync_copy(src_hbm.at[wid], srcv)
    pltpu.sync_copy(dst_hbm.at[wid], dstv)

    pltpu.sync_copy(zero_hbm.at[pl.ds(sid * RPT, RPT)],
                    acc.at[pl.ds(sid * RPT, RPT)])
    plsc.subcore_barrier()

    def body(i, carry):
        pltpu.async_copy(t_hbm.at[srcv.at[i]], rows, sem).wait()
        pltpu.sync_copy(rows, acc.at[dstv.at[i]], add=True)
        return carry

    lax.fori_loop(0, NCH, body, 0)
    plsc.subcore_barrier()

    pltpu.sync_copy(acc.at[pl.ds(sid * RPT, RPT)],
                    out_hbm.at[cid, pl.ds(sid * RPT, RPT)])


def _make_sc_agg(d):
    mesh = plsc.VectorSubcoreMesh(core_axis_name="c", subcore_axis_name="s",
                                  num_cores=NC, num_subcores=NS)
    return pl.kernel(
        _sc_agg_body,
        out_type=jax.ShapeDtypeStruct((NC, NPAD, d), jnp.float32),
        mesh=mesh,
        scratch_types=[
            pltpu.VMEM((NCH, CHUNK), jnp.int32),
            pltpu.VMEM((NCH, CHUNK), jnp.int32),
            pltpu.VMEM((CHUNK, d), jnp.float32),
            pltpu.VMEM_SHARED((NPAD, d), jnp.float32),
            pltpu.SemaphoreType.DMA,
        ],
    )


BN_ROWS = 1000


def _mm_body(x_ref, w_ref, o_ref):
    o_ref[...] = jnp.dot(x_ref[...], w_ref[...],
                         preferred_element_type=jnp.float32)


def _tc_matmul(x, w, dout):
    return pl.pallas_call(
        _mm_body,
        grid=(N // BN_ROWS,),
        in_specs=[
            pl.BlockSpec((BN_ROWS, x.shape[1]), lambda i: (i, 0)),
            pl.BlockSpec(w.shape, lambda i: (0, 0)),
        ],
        out_specs=pl.BlockSpec((BN_ROWS, dout), lambda i: (i, 0)),
        out_shape=jax.ShapeDtypeStruct((N, dout), jnp.float32),
    )(x, w)


def _stage_body(t_ref, p0_ref, p1_ref, eps_ref, b_ref, a_ref, be_ref, w_ref,
                o_ref):
    z = ((1.0 + eps_ref[0, 0]) * t_ref[...] + p0_ref[0] + p1_ref[0]
         + b_ref[...])
    h = jnp.maximum(z * a_ref[...] + be_ref[...], 0.0)
    o_ref[...] = jnp.dot(h, w_ref[...], preferred_element_type=jnp.float32)


def _tc_stage(t, p, eps, b, a, be, w, dout):
    return pl.pallas_call(
        _stage_body,
        grid=(N // BN_ROWS,),
        in_specs=[
            pl.BlockSpec((BN_ROWS, DH), lambda i: (i, 0)),
            pl.BlockSpec((1, BN_ROWS, DH), lambda i: (0, i, 0)),
            pl.BlockSpec((1, BN_ROWS, DH), lambda i: (1, i, 0)),
            pl.BlockSpec(memory_space=pltpu.SMEM),
            pl.BlockSpec((1, DH), lambda i: (0, 0)),
            pl.BlockSpec((1, DH), lambda i: (0, 0)),
            pl.BlockSpec((1, DH), lambda i: (0, 0)),
            pl.BlockSpec((DH, dout), lambda i: (0, 0)),
        ],
        out_specs=pl.BlockSpec((BN_ROWS, dout), lambda i: (i, 0)),
        out_shape=jax.ShapeDtypeStruct((N, dout), jnp.float32),
    )(t, p, p, eps, b, a, be, w)


def _final_body(h_ref, p0_ref, p1_ref, eps_ref, w_ref, b_ref, o_ref):
    hh = (1.0 + eps_ref[0, 0]) * h_ref[...] + p0_ref[0] + p1_ref[0]
    z = jnp.dot(hh, w_ref[...], preferred_element_type=jnp.float32) + b_ref[...]
    m = jnp.max(z, axis=-1, keepdims=True)
    ez = jnp.exp(z - m)
    o_ref[...] = (z - m) - jnp.log(jnp.sum(ez, axis=-1, keepdims=True))


def _tc_final(h, p, eps, w, b):
    return pl.pallas_call(
        _final_body,
        grid=(N // BN_ROWS,),
        in_specs=[
            pl.BlockSpec((BN_ROWS, DH), lambda i: (i, 0)),
            pl.BlockSpec((1, BN_ROWS, DH), lambda i: (0, i, 0)),
            pl.BlockSpec((1, BN_ROWS, DH), lambda i: (1, i, 0)),
            pl.BlockSpec(memory_space=pltpu.SMEM),
            pl.BlockSpec((DH, DOUT), lambda i: (0, 0)),
            pl.BlockSpec((1, DOUT), lambda i: (0, 0)),
        ],
        out_specs=pl.BlockSpec((BN_ROWS, DOUT), lambda i: (i, 0)),
        out_shape=jax.ShapeDtypeStruct((N, DOUT), jnp.float32),
    )(h, p, p, eps, w, b)


@jax.jit
def _run(x, edge_index, W0, b0, W1, b1, W2, b2, eps0, eps1, eps2,
         g0, be0, g1, be1):
    src3 = edge_index[0].reshape(NW, NCH, CHUNK)
    dst3 = edge_index[1].reshape(NW, NCH, CHUNK)
    zeros = jnp.zeros((NPAD, DH), jnp.float32)

    bn_s = 1.0 / jnp.sqrt(1.0 + BN_EPS)
    a0 = (g0 * bn_s).reshape(1, DH)
    a1 = (g1 * bn_s).reshape(1, DH)

    sc_agg = _make_sc_agg(DH)

    t0 = _tc_matmul(x, W0, DH)
    p0 = sc_agg(t0, src3, dst3, zeros)
    t1 = _tc_stage(t0, p0, eps0.reshape(1, 1), b0.reshape(1, DH), a0,
                   be0.reshape(1, DH), W1, DH)
    p1 = sc_agg(t1, src3, dst3, zeros)
    h2 = _tc_stage(t1, p1, eps1.reshape(1, 1), b1.reshape(1, DH), a1,
                   be1.reshape(1, DH), jnp.eye(DH, dtype=jnp.float32), DH)
    p2 = sc_agg(h2, src3, dst3, zeros)
    return _tc_final(h2, p2, eps2.reshape(1, 1), W2, b2.reshape(1, DOUT))


def kernel(x, edge_index, W0, b0, W1, b1, W2, b2, eps0, eps1, eps2,
           g0, be0, g1, be1):
    return _run(x, edge_index, W0, b0, W1, b1, W2, b2, eps0, eps1, eps2,
                g0, be0, g1, be1)


# speedup vs baseline: 12.4243x; 1.7610x over previous
import functools

import jax
import jax.numpy as jnp
from jax import lax
from jax.experimental import pallas as pl
from jax.experimental.pallas import tpu as pltpu
from jax.experimental.pallas import tpu_sc as plsc

N = 10000
E = 320000
DIN = 128
DH = 128
DOUT = 40
BN_EPS = 1e-5

NC = 2
NS = 16
NW = NC * NS
CHUNK = 128
GEC = 8
NPAD = 10240
EWP = NPAD
NCH = EWP // CHUNK
NGRP = NCH // GEC
NSUP = NGRP // 2
RPT = NPAD // NS


def _sc_agg_body(t_hbm, src_hbm, dst_hbm, out_hbm,
                 sring, dring, bufs, acc, isems, dsems, gsems):
    cid = lax.axis_index("c")
    sid = lax.axis_index("s")
    wid = sid * NC + cid

    def zrow(r, carry):
        for k in range(DH // 16):
            bufs[0][r, pl.ds(k * 16, 16)] = jnp.zeros((16,), jnp.float32)
        return carry

    lax.fori_loop(0, CHUNK, zrow, 0)
    for i in range(RPT // CHUNK):
        pltpu.sync_copy(bufs[0],
                        acc.at[pl.ds(sid * RPT + i * CHUNK, CHUNK)])
    plsc.subcore_barrier()

    def ldsrc(g, s):
        return pltpu.make_async_copy(src_hbm.at[wid, g], sring[s], isems[s])

    def lddst(g, s):
        return pltpu.make_async_copy(dst_hbm.at[wid, g], dring[s], dsems[s])

    def gth(s, t, b):
        return pltpu.make_async_copy(t_hbm.at[sring[s].at[t]], bufs[b],
                                     gsems[b])

    def sct(s, t, b):
        pltpu.sync_copy(bufs[b], acc.at[dring[s].at[t]], add=True)

    for s in range(2):
        ldsrc(s, s).start()
        lddst(s, s).start()
    ldsrc(0, 0).wait()
    gth(0, 0, 0).start()
    gth(0, 1, 1).start()
    lddst(0, 0).wait()

    def super_body(k, carry):
        g_next0 = jnp.minimum(2 * k + 2, NGRP - 1)
        g_next1 = jnp.minimum(2 * k + 3, NGRP - 1)
        for half, s in ((0, 0), (1, 1)):
            ns = 1 - s
            for t in range(GEC):
                b = t % 2
                gth(s, t, b).wait()
                sct(s, t, b)
                if t < GEC - 2:
                    gth(s, t + 2, b).start()
                else:
                    if t == GEC - 2:
                        ldsrc(0, ns).wait()
                    gth(ns, t - (GEC - 2), b).start()
            if half == 0:
                ldsrc(g_next0, 0).start()
                lddst(g_next0, 0).start()
                lddst(1, 1).wait()
        ldsrc(g_next1, 1).start()
        lddst(g_next1, 1).start()
        lddst(0, 0).wait()
        return carry

    lax.fori_loop(0, NSUP, super_body, 0)
    pltpu.make_async_copy(t_hbm.at[sring[0].at[0]], bufs[0], gsems[0]).wait()
    pltpu.make_async_copy(t_hbm.at[sring[0].at[0]], bufs[1], gsems[1]).wait()
    ldsrc(0, 1).wait()
    lddst(0, 1).wait()
    plsc.subcore_barrier()

    pltpu.sync_copy(acc.at[pl.ds(sid * RPT, RPT)],
                    out_hbm.at[cid, pl.ds(sid * RPT, RPT)])


def _make_sc_agg(d):
    mesh = plsc.VectorSubcoreMesh(core_axis_name="c", subcore_axis_name="s",
                                  num_cores=NC, num_subcores=NS)
    return pl.kernel(
        _sc_agg_body,
        out_type=jax.ShapeDtypeStruct((NC, NPAD, d), jnp.float32),
        mesh=mesh,
        scratch_types=[
            [pltpu.VMEM((GEC, CHUNK), jnp.int32) for _ in range(2)],
            [pltpu.VMEM((GEC, CHUNK), jnp.int32) for _ in range(2)],
            [pltpu.VMEM((CHUNK, d), jnp.float32) for _ in range(2)],
            pltpu.VMEM_SHARED((NPAD, d), jnp.float32),
            [pltpu.SemaphoreType.DMA for _ in range(2)],
            [pltpu.SemaphoreType.DMA for _ in range(2)],
            [pltpu.SemaphoreType.DMA for _ in range(2)],
        ],
    )


BN_ROWS = 1000


def _mm_body(x_ref, w_ref, o_ref):
    o_ref[...] = jnp.dot(x_ref[...], w_ref[...],
                         preferred_element_type=jnp.float32)


def _tc_matmul(x, w, dout):
    return pl.pallas_call(
        _mm_body,
        grid=(N // BN_ROWS,),
        in_specs=[
            pl.BlockSpec((BN_ROWS, x.shape[1]), lambda i: (i, 0)),
            pl.BlockSpec(w.shape, lambda i: (0, 0)),
        ],
        out_specs=pl.BlockSpec((BN_ROWS, dout), lambda i: (i, 0)),
        out_shape=jax.ShapeDtypeStruct((N, dout), jnp.float32),
    )(x, w)


def _stage_body(t_ref, p0_ref, p1_ref, eps_ref, b_ref, a_ref, be_ref, w_ref,
                o_ref):
    z = ((1.0 + eps_ref[0, 0]) * t_ref[...] + p0_ref[0] + p1_ref[0]
         + b_ref[...])
    h = jnp.maximum(z * a_ref[...] + be_ref[...], 0.0)
    o_ref[...] = jnp.dot(h, w_ref[...], preferred_element_type=jnp.float32)


def _tc_stage(t, p, eps, b, a, be, w, dout):
    return pl.pallas_call(
        _stage_body,
        grid=(N // BN_ROWS,),
        in_specs=[
            pl.BlockSpec((BN_ROWS, DH), lambda i: (i, 0)),
            pl.BlockSpec((1, BN_ROWS, DH), lambda i: (0, i, 0)),
            pl.BlockSpec((1, BN_ROWS, DH), lambda i: (1, i, 0)),
            pl.BlockSpec(memory_space=pltpu.SMEM),
            pl.BlockSpec((1, DH), lambda i: (0, 0)),
            pl.BlockSpec((1, DH), lambda i: (0, 0)),
            pl.BlockSpec((1, DH), lambda i: (0, 0)),
            pl.BlockSpec((DH, dout), lambda i: (0, 0)),
        ],
        out_specs=pl.BlockSpec((BN_ROWS, dout), lambda i: (i, 0)),
        out_shape=jax.ShapeDtypeStruct((N, dout), jnp.float32),
    )(t, p, p, eps, b, a, be, w)


def _final_body(t_ref, p0_ref, p1_ref, eps_ref, b_ref, o_ref):
    z = ((1.0 + eps_ref[0, 0]) * t_ref[...] + p0_ref[0] + p1_ref[0]
         + b_ref[...])
    m = jnp.max(z, axis=-1, keepdims=True)
    ez = jnp.exp(z - m)
    o_ref[...] = (z - m) - jnp.log(jnp.sum(ez, axis=-1, keepdims=True))


def _tc_final(t, p, eps, b):
    return pl.pallas_call(
        _final_body,
        grid=(N // BN_ROWS,),
        in_specs=[
            pl.BlockSpec((BN_ROWS, DOUT), lambda i: (i, 0)),
            pl.BlockSpec((1, BN_ROWS, DOUT), lambda i: (0, i, 0)),
            pl.BlockSpec((1, BN_ROWS, DOUT), lambda i: (1, i, 0)),
            pl.BlockSpec(memory_space=pltpu.SMEM),
            pl.BlockSpec((1, DOUT), lambda i: (0, 0)),
        ],
        out_specs=pl.BlockSpec((BN_ROWS, DOUT), lambda i: (i, 0)),
        out_shape=jax.ShapeDtypeStruct((N, DOUT), jnp.float32),
    )(t, p, p, eps, b)


@jax.jit
def _run(x, edge_index, W0, b0, W1, b1, W2, b2, eps0, eps1, eps2,
         g0, be0, g1, be1):
    npad_e = NW * EWP - E
    pad_src = jnp.arange(npad_e, dtype=jnp.int32) % N
    pad_dst = N + (jnp.arange(npad_e, dtype=jnp.int32) % (NPAD - N))
    src3 = jnp.concatenate([edge_index[0], pad_src]).reshape(
        NW, NGRP, GEC, CHUNK)
    dst3 = jnp.concatenate([edge_index[1], pad_dst]).reshape(
        NW, NGRP, GEC, CHUNK)

    bn_s = 1.0 / jnp.sqrt(1.0 + BN_EPS)
    a0 = (g0 * bn_s).reshape(1, DH)
    a1 = (g1 * bn_s).reshape(1, DH)

    sc_agg = _make_sc_agg(DH)

    t0 = _tc_matmul(x, W0, DH)
    p0 = sc_agg(t0, src3, dst3)
    t1 = _tc_stage(t0, p0, eps0.reshape(1, 1), b0.reshape(1, DH), a0,
                   be0.reshape(1, DH), W1, DH)
    p1 = sc_agg(t1, src3, dst3)
    t2p = _tc_stage(t1, p1, eps1.reshape(1, 1), b1.reshape(1, DH), a1,
                    be1.reshape(1, DH), jnp.pad(W2, ((0, 0), (0, DH - DOUT))),
                    DH)
    p2 = sc_agg(t2p, src3, dst3)
    return _tc_final(t2p[:, :DOUT], p2[:, :, :DOUT], eps2.reshape(1, 1),
                     b2.reshape(1, DOUT))


def kernel(x, edge_index, W0, b0, W1, b1, W2, b2, eps0, eps1, eps2,
           g0, be0, g1, be1):
    return _run(x, edge_index, W0, b0, W1, b1, W2, b2, eps0, eps1, eps2,
                g0, be0, g1, be1)
